# baseline (speedup 1.0000x reference)
import functools

import jax
import jax.numpy as jnp
from jax import lax
from jax.experimental import pallas as pl
from jax.experimental.pallas import tpu as pltpu
from jax.experimental.pallas import tpu_sc as plsc

NPAD = 10240
D = 128
NT = 32
NS = 16
NPT = NPAD // NT
CAP = 14336
CAPB = CAP + 128
CH = 2560
GB2 = 128
GBB = 64
ACC_ROWS = NS * NPT + 1
DUMP_SP = NS * NPT


def _sc_filter_body(src_hbm, dst_hbm, ew_hbm,
                    srcl, ewl, dstll, ecnt, deg, cntv,
                    srcb, ewb, dstlb, degl, cntl,
                    dstst, srcst, ewst, ebuf):
    c = lax.axis_index("c")
    s = lax.axis_index("s")
    w = s * 2 + c
    lo = w * NPT
    E = src_hbm.shape[0]
    ones16 = jnp.ones((16,), jnp.float32)

    def _z(i, _):
        degl[pl.ds(i * 16, 16)] = jnp.zeros((16,), jnp.float32)
        cntl[pl.ds(i * 16, 16)] = jnp.zeros((16,), jnp.float32)
        return 0
    lax.fori_loop(0, NPT // 16, _z, 0)

    def _chunk(ci, cnt):
        pltpu.sync_copy(src_hbm.at[pl.ds(ci * CH, CH)], srcst)
        pltpu.sync_copy(dst_hbm.at[pl.ds(ci * CH, CH)], dstst)
        pltpu.sync_copy(ew_hbm.at[pl.ds(ci * CH, CH)], ewst)

        def _g(gi, cnt):
            d16 = dstst[pl.ds(gi * 16, 16)]
            s16 = srcst[pl.ds(gi * 16, 16)]
            e16 = ewst[pl.ds(gi * 16, 16)]
            dl16 = d16 - lo
            own = (d16 >= lo) & (dl16 < NPT) & (cnt <= CAP - 16)
            base = jnp.minimum(cnt, CAP - 16)
            plsc.store_compressed(srcb.at[pl.ds(base, 16)], s16, mask=own)
            plsc.store_compressed(ewb.at[pl.ds(base, 16)], e16, mask=own)
            plsc.store_compressed(dstlb.at[pl.ds(base, 16)], dl16, mask=own)
            plsc.addupdate_scatter(degl, [dl16], e16, mask=own)
            plsc.addupdate_scatter(cntl, [dl16], ones16, mask=own)
            return cnt + jnp.sum(jnp.where(own, 1, 0))

        return lax.fori_loop(0, CH // 16, _g, cnt)

    cnt = lax.fori_loop(0, E // CH, _chunk, jnp.int32(0))

    def _pad(i, _):
        srcb[pl.ds(cnt + i * 16, 16)] = jnp.zeros((16,), jnp.int32)
        ewb[pl.ds(cnt + i * 16, 16)] = jnp.zeros((16,), jnp.float32)
        dstlb[pl.ds(cnt + i * 16, 16)] = jnp.full((16,), NPT, jnp.int32)
        return 0
    lax.fori_loop(0, GB2 // 16, _pad, 0)
    cntp = ((cnt + GB2 - 1) // GB2) * GB2

    ebuf[pl.ds(0, 16)] = jnp.full((16,), cntp, jnp.int32)
    pltpu.sync_copy(ebuf, ecnt.at[pl.ds(w * 16, 16)])
    pltpu.sync_copy(srcb.at[pl.ds(0, CAP)], srcl.at[pl.ds(w * CAP, CAP)])
    pltpu.sync_copy(ewb.at[pl.ds(0, CAP)], ewl.at[pl.ds(w * CAP, CAP)])
    pltpu.sync_copy(dstlb.at[pl.ds(0, CAP)], dstll.at[pl.ds(w * CAP, CAP)])
    pltpu.sync_copy(degl, deg.at[pl.ds(lo, NPT)])
    pltpu.sync_copy(cntl, cntv.at[pl.ds(lo, NPT)])


def _sc_max_s1_body(srcl, dstll, ecnt, x_hbm,
                    mxf, s1_out,
                    srcb, dstlb, maxa, gbuf, idxb, ebuf, zbuf, acc1, sem):
    c = lax.axis_index("c")
    s = lax.axis_index("s")
    w = s * 2 + c
    lo = w * NPT

    def _init(i, _):
        maxa[pl.ds(i * 16, 16)] = jnp.full((16,), -3.0e38, jnp.float32)
        return 0
    lax.fori_loop(0, (NPT * D + D) // 16, _init, 0)

    def _zrow(i, _):
        for k in range(D // 16):
            zbuf[i, pl.ds(k * 16, 16)] = jnp.zeros((16,), jnp.float32)
        return 0
    lax.fori_loop(0, 40, _zrow, 0)
    for r in range(NPT // 40):
        pltpu.sync_copy(zbuf, acc1.at[pl.ds(s * NPT + r * 40, 40), :])

    @pl.when(s == 0)
    def _():
        pltpu.sync_copy(zbuf.at[pl.ds(0, 1), :], acc1.at[pl.ds(DUMP_SP, 1), :])

    plsc.subcore_barrier()

    pltpu.sync_copy(ecnt.at[pl.ds(w * 16, 16)], ebuf)
    n_edges = ebuf[pl.ds(0, 16)][0]

    def _batch(b, _):
        pltpu.sync_copy(srcl.at[pl.ds(w * CAP + b * GB2, GB2)],
                        srcb.at[pl.ds(0, GB2)])
        pltpu.sync_copy(dstll.at[pl.ds(w * CAP + b * GB2, GB2)],
                        dstlb.at[pl.ds(0, GB2)])
        pltpu.async_copy(x_hbm.at[srcb.at[pl.ds(0, GB2)]], gbuf, sem).wait()

        for g in range(GB2 // 16):
            dl16 = dstlb[pl.ds(g * 16, 16)]
            idxb[pl.ds(g * 16, 16)] = jnp.where(dl16 >= NPT, DUMP_SP,
                                                s * NPT + dl16)

        def _edge(j, _):
            dl = dstlb[pl.ds(j, 16)][0]
            rb = dl * D
            for k in range(D // 16):
                r = gbuf[j, pl.ds(k * 16, 16)]
                m0 = maxa[pl.ds(rb + k * 16, 16)]
                maxa[pl.ds(rb + k * 16, 16)] = jnp.maximum(m0, r)
            return 0

        lax.fori_loop(0, GB2, _edge, 0)
        pltpu.sync_copy(gbuf, acc1.at[idxb], add=True)
        return 0

    lax.fori_loop(0, n_edges // GB2, _batch, 0)

    pltpu.sync_copy(maxa.at[pl.ds(0, NPT * D)],
                    mxf.at[pl.ds(w * NPT * D, NPT * D)])

    plsc.subcore_barrier()
    for r in range(NPT // 40):
        pltpu.sync_copy(acc1.at[pl.ds(s * NPT + r * 40, 40), :], zbuf)
        pltpu.sync_copy(zbuf, s1_out.at[pl.ds(lo + r * 40, 40), :])


def _sc_sums_body(srcl, ewl, dstll, ecnt, dinv_hbm, x_hbm,
                  sa_out, sew_out,
                  srcb, ewb, dstlb, dinvb, gbuf, sab, sewb, idxb, abuf, ebuf,
                  zbuf, acca, accew, sem):
    c = lax.axis_index("c")
    s = lax.axis_index("s")
    w = s * 2 + c
    lo = w * NPT

    def _zrow(i, _):
        for k in range(D // 16):
            zbuf[i, pl.ds(k * 16, 16)] = jnp.zeros((16,), jnp.float32)
        return 0
    lax.fori_loop(0, 40, _zrow, 0)
    for r in range(NPT // 40):
        sl = pl.ds(s * NPT + r * 40, 40)
        pltpu.sync_copy(zbuf, acca.at[sl, :])
        pltpu.sync_copy(zbuf, accew.at[sl, :])

    @pl.when(s == 0)
    def _():
        dsl = pl.ds(DUMP_SP, 1)
        pltpu.sync_copy(zbuf.at[pl.ds(0, 1), :], acca.at[dsl, :])
        pltpu.sync_copy(zbuf.at[pl.ds(0, 1), :], accew.at[dsl, :])

    plsc.subcore_barrier()

    pltpu.sync_copy(dinv_hbm, dinvb)
    pltpu.sync_copy(ecnt.at[pl.ds(w * 16, 16)], ebuf)
    n_edges = ebuf[pl.ds(0, 16)][0]

    def _batch(b, _):
        pltpu.sync_copy(srcl.at[pl.ds(w * CAP + b * GBB, GBB)],
                        srcb.at[pl.ds(0, GBB)])
        pltpu.sync_copy(ewl.at[pl.ds(w * CAP + b * GBB, GBB)],
                        ewb.at[pl.ds(0, GBB)])
        pltpu.sync_copy(dstll.at[pl.ds(w * CAP + b * GBB, GBB)],
                        dstlb.at[pl.ds(0, GBB)])
        pltpu.async_copy(x_hbm.at[srcb.at[pl.ds(0, GBB)]], gbuf, sem).wait()

        for g in range(GBB // 16):
            s16 = srcb[pl.ds(g * 16, 16)]
            dl16 = dstlb[pl.ds(g * 16, 16)]
            e16 = ewb[pl.ds(g * 16, 16)]
            dv16 = plsc.load_gather(dinvb, [s16])
            abuf[pl.ds(g * 16, 16)] = dv16 * e16
            idxb[pl.ds(g * 16, 16)] = jnp.where(dl16 >= NPT, DUMP_SP,
                                                s * NPT + dl16)

        def _edge(j, _):
            a = abuf[pl.ds(j, 16)][0]
            e = ewb[pl.ds(j, 16)][0]
            for k in range(D // 16):
                r = gbuf[j, pl.ds(k * 16, 16)]
                sab[j, pl.ds(k * 16, 16)] = r * a
                sewb[j, pl.ds(k * 16, 16)] = r * e
            return 0

        lax.fori_loop(0, GBB, _edge, 0)

        pltpu.sync_copy(sab, acca.at[idxb], add=True)
        pltpu.sync_copy(sewb, accew.at[idxb], add=True)
        return 0

    lax.fori_loop(0, n_edges // GBB, _batch, 0)

    plsc.subcore_barrier()
    for r in range(NPT // 40):
        ssl = pl.ds(s * NPT + r * 40, 40)
        osl = pl.ds(lo + r * 40, 40)
        pltpu.sync_copy(acca.at[ssl, :], zbuf)
        pltpu.sync_copy(zbuf, sa_out.at[osl, :])
        pltpu.sync_copy(accew.at[ssl, :], zbuf)
        pltpu.sync_copy(zbuf, sew_out.at[osl, :])


def _tc_scalars_body(deg_ref, cnt_ref, dinv_ref, idg_ref, has_ref):
    dg = jnp.clip(deg_ref[...], 1e-9, None)
    dinv_ref[...] = lax.rsqrt(dg)
    idg_ref[...] = 1.0 / dg
    has_ref[...] = (cnt_ref[...] > 0).astype(jnp.float32)


def _tc_mix_body(wts_ref, eps_ref, x_ref, sa_ref, sew_ref, s1_ref, m_ref,
                 dinv_ref, idg_ref, has_ref,
                 wgcn, bgcn, wss, wsn, bsage, wms, wmn, bsmax,
                 wg1, bg1, wg2, bg2, wlin, blin, out_ref):
    w0 = wts_ref[0]
    w1 = wts_ref[1]
    w2 = wts_ref[2]
    w3 = wts_ref[3]
    w4 = wts_ref[4]
    ep = eps_ref[0]
    xv = x_ref[...]
    f32 = jnp.float32
    gcn = jnp.dot(dinv_ref[...] * sa_ref[...], wgcn[...],
                  preferred_element_type=f32) + bgcn[...]
    sage = (jnp.dot(xv, wss[...], preferred_element_type=f32)
            + jnp.dot(idg_ref[...] * sew_ref[...], wsn[...],
                      preferred_element_type=f32)
            + bsage[...])
    mxv = jnp.where(has_ref[...] > 0, m_ref[...], 0.0)
    smax = (jnp.dot(xv, wms[...], preferred_element_type=f32)
            + jnp.dot(mxv, wmn[...], preferred_element_type=f32) + bsmax[...])
    hg = (1.0 + ep) * xv + s1_ref[...]
    h1 = jax.nn.relu(jnp.dot(hg, wg1[...], preferred_element_type=f32)
                     + bg1[...])
    gin = jnp.dot(h1, wg2[...], preferred_element_type=f32) + bg2[...]
    lin = jnp.dot(xv, wlin[...], preferred_element_type=f32) + blin[...]
    out_ref[...] = w0 * gcn + w1 * sage + w2 * smax + w3 * gin + w4 * lin


@functools.cache
def _build_sc_kernels():
    mesh = plsc.VectorSubcoreMesh(core_axis_name="c", subcore_axis_name="s",
                                  num_cores=2, num_subcores=16)
    cparams = pltpu.CompilerParams(needs_layout_passes=False)

    sc_filter = pl.kernel(
        _sc_filter_body,
        out_type=[
            jax.ShapeDtypeStruct((NT * CAP,), jnp.int32),
            jax.ShapeDtypeStruct((NT * CAP,), jnp.float32),
            jax.ShapeDtypeStruct((NT * CAP,), jnp.int32),
            jax.ShapeDtypeStruct((NT * 16,), jnp.int32),
            jax.ShapeDtypeStruct((NPAD,), jnp.float32),
            jax.ShapeDtypeStruct((NPAD,), jnp.float32),
        ],
        mesh=mesh,
        compiler_params=cparams,
        scratch_types=[
            pltpu.VMEM((CAPB,), jnp.int32),
            pltpu.VMEM((CAPB,), jnp.float32),
            pltpu.VMEM((CAPB,), jnp.int32),
            pltpu.VMEM((NPT,), jnp.float32),
            pltpu.VMEM((NPT,), jnp.float32),
            pltpu.VMEM((CH,), jnp.int32),
            pltpu.VMEM((CH,), jnp.int32),
            pltpu.VMEM((CH,), jnp.float32),
            pltpu.VMEM((16,), jnp.int32),
        ],
    )

    sc_max_s1 = pl.kernel(
        _sc_max_s1_body,
        out_type=[
            jax.ShapeDtypeStruct((NPAD * D,), jnp.float32),
            jax.ShapeDtypeStruct((NPAD, D), jnp.float32),
        ],
        mesh=mesh,
        compiler_params=cparams,
        scratch_types=[
            pltpu.VMEM((GB2 + 16,), jnp.int32),
            pltpu.VMEM((GB2 + 16,), jnp.int32),
            pltpu.VMEM((NPT * D + D,), jnp.float32),
            pltpu.VMEM((GB2, D), jnp.float32),
            pltpu.VMEM((GB2,), jnp.int32),
            pltpu.VMEM((16,), jnp.int32),
            pltpu.VMEM((40, D), jnp.float32),
            pltpu.VMEM_SHARED((ACC_ROWS, D), jnp.float32),
            pltpu.SemaphoreType.DMA,
        ],
    )

    sc_sums = pl.kernel(
        _sc_sums_body,
        out_type=[
            jax.ShapeDtypeStruct((NPAD, D), jnp.float32),
            jax.ShapeDtypeStruct((NPAD, D), jnp.float32),
        ],
        mesh=mesh,
        compiler_params=cparams,
        scratch_types=[
            pltpu.VMEM((GBB + 16,), jnp.int32),
            pltpu.VMEM((GBB + 16,), jnp.float32),
            pltpu.VMEM((GBB + 16,), jnp.int32),
            pltpu.VMEM((NPAD,), jnp.float32),
            pltpu.VMEM((GBB, D), jnp.float32),
            pltpu.VMEM((GBB, D), jnp.float32),
            pltpu.VMEM((GBB, D), jnp.float32),
            pltpu.VMEM((GBB,), jnp.int32),
            pltpu.VMEM((GBB + 16,), jnp.float32),
            pltpu.VMEM((16,), jnp.int32),
            pltpu.VMEM((40, D), jnp.float32),
            pltpu.VMEM_SHARED((ACC_ROWS, D), jnp.float32),
            pltpu.VMEM_SHARED((ACC_ROWS, D), jnp.float32),
            pltpu.SemaphoreType.DMA,
        ],
    )
    return sc_filter, sc_max_s1, sc_sums


def kernel(x, weights, edge_index, edge_weights, edge_attr, W_gcn, b_gcn,
           W_sage_self, W_sage_neigh, b_sage, W_smax_self, W_smax_neigh,
           b_smax, W_gin1, b_gin1, W_gin2, b_gin2, eps, W_lin, b_lin,
           with_linear):
    n, d = x.shape
    xp = jnp.pad(x, ((0, NPAD - n), (0, 0)))
    src = edge_index[0]
    dst = edge_index[1]

    sc_filter, sc_max_s1, sc_sums = _build_sc_kernels()
    srcl, ewl, dstll, ecnt, deg, cntv = sc_filter(src, dst, edge_weights)
    mxf, s1 = sc_max_s1(srcl, dstll, ecnt, xp)

    dinv2, idg2, has2 = pl.pallas_call(
        _tc_scalars_body,
        grid=(1,),
        in_specs=[pl.BlockSpec((NPAD // D, D), lambda i: (0, 0)),
                  pl.BlockSpec((NPAD // D, D), lambda i: (0, 0))],
        out_specs=[pl.BlockSpec((NPAD // D, D), lambda i: (0, 0))] * 3,
        out_shape=[jax.ShapeDtypeStruct((NPAD // D, D), jnp.float32)] * 3,
    )(deg.reshape(NPAD // D, D), cntv.reshape(NPAD // D, D))

    sa, sew = sc_sums(srcl, ewl, dstll, ecnt, dinv2.reshape(NPAD), xp)

    nblk = NPAD // D
    row_spec = pl.BlockSpec((D, D), lambda i: (i, 0))
    col_spec = pl.BlockSpec((D, 1), lambda i: (i, 0))
    full_spec = pl.BlockSpec((D, D), lambda i: (0, 0))
    bias_spec = pl.BlockSpec((1, D), lambda i: (0, 0))
    smem_spec = pl.BlockSpec(memory_space=pltpu.SMEM)
    out = pl.pallas_call(
        _tc_mix_body,
        grid=(nblk,),
        in_specs=[smem_spec, smem_spec,
                  row_spec, row_spec, row_spec, row_spec, row_spec,
                  col_spec, col_spec, col_spec,
                  full_spec, bias_spec, full_spec, full_spec, bias_spec,
                  full_spec, full_spec, bias_spec,
                  full_spec, bias_spec, full_spec, bias_spec,
                  full_spec, bias_spec],
        out_specs=pl.BlockSpec((D, D), lambda i: (i, 0)),
        out_shape=jax.ShapeDtypeStruct((NPAD, D), jnp.float32),
    )(weights, eps.reshape(1), xp, sa, sew, s1, mxf.reshape(NPAD, D),
      dinv2.reshape(NPAD, 1), idg2.reshape(NPAD, 1), has2.reshape(NPAD, 1),
      W_gcn, b_gcn.reshape(1, D), W_sage_self, W_sage_neigh,
      b_sage.reshape(1, D), W_smax_self, W_smax_neigh, b_smax.reshape(1, D),
      W_gin1, b_gin1.reshape(1, D), W_gin2, b_gin2.reshape(1, D),
      W_lin, b_lin.reshape(1, D))
    return out[:n]


# speedup vs baseline: 1.2488x; 1.2488x over previous
import functools

import jax
import jax.numpy as jnp
from jax import lax
from jax.experimental import pallas as pl
from jax.experimental.pallas import tpu as pltpu
from jax.experimental.pallas import tpu_sc as plsc

NPAD = 10240
D = 128
NT = 32
NS = 16
NPT = NPAD // NT
CAP = 14336
CAPB = CAP + 256
CH = 4000
GB2 = 128
GBB = 64
ACC_ROWS = NS * NPT + 1
DUMP_SP = NS * NPT


def _sc_filter_body(src_hbm, dst_hbm, ew_hbm,
                    srcl, ewl, dstll, ecnt, deg, cntv,
                    srcb, ewb, dstlb, degl, cntl,
                    dstst, srcst, ewst, ebuf, sema, semb):
    c = lax.axis_index("c")
    s = lax.axis_index("s")
    w = s * 2 + c
    lo = w * NPT
    E = src_hbm.shape[0]
    NCH = E // CH
    ones16 = jnp.ones((16,), jnp.float32)
    sems = (sema, semb)

    def _z(i, _):
        degl[pl.ds(i * 16, 16)] = jnp.zeros((16,), jnp.float32)
        cntl[pl.ds(i * 16, 16)] = jnp.zeros((16,), jnp.float32)
        return 0
    lax.fori_loop(0, NPT // 16, _z, 0)

    def _issue(ci, p):
        sl = pl.ds(ci * CH, CH)
        dl = pl.ds(p * CH, CH)
        pltpu.async_copy(src_hbm.at[sl], srcst.at[dl], sems[p])
        pltpu.async_copy(dst_hbm.at[sl], dstst.at[dl], sems[p])
        pltpu.async_copy(ew_hbm.at[sl], ewst.at[dl], sems[p])

    def _drain(p):
        sl = pl.ds(0, CH)
        dl = pl.ds(p * CH, CH)
        pltpu.make_async_copy(src_hbm.at[sl], srcst.at[dl], sems[p]).wait()
        pltpu.make_async_copy(dst_hbm.at[sl], dstst.at[dl], sems[p]).wait()
        pltpu.make_async_copy(ew_hbm.at[sl], ewst.at[dl], sems[p]).wait()

    def _process(p, cnt):
        def _g(gi, cnt):
            d16 = dstst[pl.ds(p * CH + gi * 16, 16)]
            s16 = srcst[pl.ds(p * CH + gi * 16, 16)]
            e16 = ewst[pl.ds(p * CH + gi * 16, 16)]
            dl16 = d16 - lo
            own = (d16 >= lo) & (dl16 < NPT) & (cnt <= CAP - 16)
            base = jnp.minimum(cnt, CAP - 16)
            plsc.store_compressed(srcb.at[pl.ds(base, 16)], s16, mask=own)
            plsc.store_compressed(ewb.at[pl.ds(base, 16)], e16, mask=own)
            plsc.store_compressed(dstlb.at[pl.ds(base, 16)], dl16, mask=own)
            plsc.addupdate_scatter(degl, [dl16], e16, mask=own)
            plsc.addupdate_scatter(cntl, [dl16], ones16, mask=own)
            return cnt + jnp.sum(jnp.where(own, 1, 0))
        return lax.fori_loop(0, CH // 16, _g, cnt)

    _issue(0, 0)

    def _loop(i2, cnt):
        ci = 2 * i2
        _issue(ci + 1, 1)
        _drain(0)
        cnt = _process(0, cnt)

        @pl.when(ci + 2 < NCH)
        def _():
            _issue(ci + 2, 0)

        _drain(1)
        cnt = _process(1, cnt)
        return cnt

    cnt = lax.fori_loop(0, NCH // 2, _loop, jnp.int32(0))

    def _pad(i, _):
        srcb[pl.ds(cnt + i * 16, 16)] = jnp.zeros((16,), jnp.int32)
        ewb[pl.ds(cnt + i * 16, 16)] = jnp.zeros((16,), jnp.float32)
        dstlb[pl.ds(cnt + i * 16, 16)] = jnp.full((16,), NPT, jnp.int32)
        return 0
    lax.fori_loop(0, 256 // 16, _pad, 0)
    cntp = jnp.maximum(((cnt + 255) // 256) * 256, 256)

    ebuf[pl.ds(0, 16)] = jnp.full((16,), cntp, jnp.int32)
    pltpu.sync_copy(ebuf, ecnt.at[pl.ds(w * 16, 16)])
    pltpu.sync_copy(srcb.at[pl.ds(0, CAP)], srcl.at[pl.ds(w * CAP, CAP)])
    pltpu.sync_copy(ewb.at[pl.ds(0, CAP)], ewl.at[pl.ds(w * CAP, CAP)])
    pltpu.sync_copy(dstlb.at[pl.ds(0, CAP)], dstll.at[pl.ds(w * CAP, CAP)])
    pltpu.sync_copy(degl, deg.at[pl.ds(lo, NPT)])
    pltpu.sync_copy(cntl, cntv.at[pl.ds(lo, NPT)])


def _sc_max_s1_body(srcl, dstll, ecnt, x_hbm,
                    mxf, s1_out,
                    srcb, dstlb, maxa, gbuf, idxb, ebuf, zbuf, acc1,
                    semg0, semg1, sems0, sems1):
    c = lax.axis_index("c")
    s = lax.axis_index("s")
    w = s * 2 + c
    lo = w * NPT
    semg = (semg0, semg1)
    sems = (sems0, sems1)

    def _init(i, _):
        maxa[pl.ds(i * 16, 16)] = jnp.full((16,), -3.0e38, jnp.float32)
        return 0
    lax.fori_loop(0, (NPT * D + D) // 16, _init, 0)

    def _zrow(i, _):
        for k in range(D // 16):
            zbuf[i, pl.ds(k * 16, 16)] = jnp.zeros((16,), jnp.float32)
        return 0
    lax.fori_loop(0, 40, _zrow, 0)
    for r in range(NPT // 40):
        pltpu.sync_copy(zbuf, acc1.at[pl.ds(s * NPT + r * 40, 40), :])

    @pl.when(s == 0)
    def _():
        pltpu.sync_copy(zbuf.at[pl.ds(0, 1), :], acc1.at[pl.ds(DUMP_SP, 1), :])

    plsc.subcore_barrier()

    pltpu.sync_copy(ecnt.at[pl.ds(w * 16, 16)], ebuf)
    n_edges = ebuf[pl.ds(0, 16)][0]
    nb = n_edges // GB2

    def _lists(b, p):
        sl = pl.ds(w * CAP + b * GB2, GB2)
        pltpu.sync_copy(srcl.at[sl], srcb.at[p, pl.ds(0, GB2)])
        pltpu.sync_copy(dstll.at[sl], dstlb.at[p, pl.ds(0, GB2)])

    def _issue_gather(p):
        pltpu.async_copy(x_hbm.at[srcb.at[p, pl.ds(0, GB2)]], gbuf.at[p],
                         semg[p])

    def _drain_gather(p):
        pltpu.make_async_copy(x_hbm.at[srcb.at[p, pl.ds(0, GB2)]],
                              gbuf.at[p], semg[p]).wait()

    def _compute(b, p):
        for g in range(GB2 // 16):
            dl16 = dstlb[p, pl.ds(g * 16, 16)]
            idxb[p, pl.ds(g * 16, 16)] = jnp.where(dl16 >= NPT, DUMP_SP,
                                                   s * NPT + dl16)

        d = pltpu.async_copy(gbuf.at[p], acc1.at[idxb.at[p]], sems[p],
                             add=True)

        def _edge(j, _):
            dl = dstlb[p, pl.ds(j, 16)][0]
            rb = dl * D
            for k in range(D // 16):
                r = gbuf[p, j, pl.ds(k * 16, 16)]
                m0 = maxa[pl.ds(rb + k * 16, 16)]
                maxa[pl.ds(rb + k * 16, 16)] = jnp.maximum(m0, r)
            return 0

        lax.fori_loop(0, GB2, _edge, 0)
        return d

    _lists(0, 0)
    _issue_gather(0)

    def _loop(i2, _):
        b = 2 * i2
        _lists(b + 1, 1)
        _issue_gather(1)
        _drain_gather(0)
        d0 = _compute(b, 0)
        d0.wait()

        @pl.when(b + 2 < nb)
        def _():
            _lists(b + 2, 0)
            _issue_gather(0)

        _drain_gather(1)
        d1 = _compute(b + 1, 1)
        d1.wait()
        return 0

    lax.fori_loop(0, nb // 2, _loop, 0)

    pltpu.sync_copy(maxa.at[pl.ds(0, NPT * D)],
                    mxf.at[pl.ds(w * NPT * D, NPT * D)])

    plsc.subcore_barrier()
    for r in range(NPT // 40):
        pltpu.sync_copy(acc1.at[pl.ds(s * NPT + r * 40, 40), :], zbuf)
        pltpu.sync_copy(zbuf, s1_out.at[pl.ds(lo + r * 40, 40), :])


def _sc_sums_body(srcl, ewl, dstll, ecnt, dinv_hbm, x_hbm,
                  sa_out, sew_out,
                  srcb, ewb, dstlb, dvbuf, gbuf, sab, sewb, idxb, abuf, ebuf,
                  zbuf, acca, accew, semg0, semg1, semsc):
    c = lax.axis_index("c")
    s = lax.axis_index("s")
    w = s * 2 + c
    lo = w * NPT
    semg = (semg0, semg1)

    def _zrow(i, _):
        for k in range(D // 16):
            zbuf[i, pl.ds(k * 16, 16)] = jnp.zeros((16,), jnp.float32)
        return 0
    lax.fori_loop(0, 40, _zrow, 0)
    for r in range(NPT // 40):
        sl = pl.ds(s * NPT + r * 40, 40)
        pltpu.sync_copy(zbuf, acca.at[sl, :])
        pltpu.sync_copy(zbuf, accew.at[sl, :])

    @pl.when(s == 0)
    def _():
        dsl = pl.ds(DUMP_SP, 1)
        pltpu.sync_copy(zbuf.at[pl.ds(0, 1), :], acca.at[dsl, :])
        pltpu.sync_copy(zbuf.at[pl.ds(0, 1), :], accew.at[dsl, :])

    plsc.subcore_barrier()

    pltpu.sync_copy(ecnt.at[pl.ds(w * 16, 16)], ebuf)
    n_edges = ebuf[pl.ds(0, 16)][0]
    nb = n_edges // GBB

    def _lists(b, p):
        sl = pl.ds(w * CAP + b * GBB, GBB)
        pltpu.sync_copy(srcl.at[sl], srcb.at[p, pl.ds(0, GBB)])
        pltpu.sync_copy(ewl.at[sl], ewb.at[p, pl.ds(0, GBB)])
        pltpu.sync_copy(dstll.at[sl], dstlb.at[p, pl.ds(0, GBB)])

    def _issue_gather(p):
        idx = srcb.at[p, pl.ds(0, GBB)]
        pltpu.async_copy(x_hbm.at[idx], gbuf.at[p], semg[p])
        pltpu.async_copy(dinv_hbm.at[idx], dvbuf.at[p, pl.ds(0, GBB)], semg[p])

    def _drain_gather(p):
        idx = srcb.at[p, pl.ds(0, GBB)]
        pltpu.make_async_copy(x_hbm.at[idx], gbuf.at[p], semg[p]).wait()
        pltpu.make_async_copy(dinv_hbm.at[idx], dvbuf.at[p, pl.ds(0, GBB)],
                              semg[p]).wait()

    def _compute(b, p):
        for g in range(GBB // 16):
            dl16 = dstlb[p, pl.ds(g * 16, 16)]
            e16 = ewb[p, pl.ds(g * 16, 16)]
            dv16 = dvbuf[p, pl.ds(g * 16, 16)]
            abuf[p, pl.ds(g * 16, 16)] = dv16 * e16
            idxb[p, pl.ds(g * 16, 16)] = jnp.where(dl16 >= NPT, DUMP_SP,
                                                   s * NPT + dl16)

        def _edge(j, _):
            a = abuf[p, pl.ds(j, 16)][0]
            e = ewb[p, pl.ds(j, 16)][0]
            for k in range(D // 16):
                r = gbuf[p, j, pl.ds(k * 16, 16)]
                sab[j, pl.ds(k * 16, 16)] = r * a
                sewb[j, pl.ds(k * 16, 16)] = r * e
            return 0

        lax.fori_loop(0, GBB, _edge, 0)
        da = pltpu.async_copy(sab, acca.at[idxb.at[p]], semsc, add=True)
        de = pltpu.async_copy(sewb, accew.at[idxb.at[p]], semsc, add=True)
        return da, de

    _lists(0, 0)
    _issue_gather(0)

    def _loop(i2, _):
        b = 2 * i2
        _lists(b + 1, 1)
        _issue_gather(1)
        _drain_gather(0)
        da, de = _compute(b, 0)

        @pl.when(b + 2 < nb)
        def _():
            _lists(b + 2, 0)
            _issue_gather(0)

        _drain_gather(1)
        da.wait()
        de.wait()
        da, de = _compute(b + 1, 1)
        da.wait()
        de.wait()
        return 0

    lax.fori_loop(0, nb // 2, _loop, 0)

    plsc.subcore_barrier()
    for r in range(NPT // 40):
        ssl = pl.ds(s * NPT + r * 40, 40)
        osl = pl.ds(lo + r * 40, 40)
        pltpu.sync_copy(acca.at[ssl, :], zbuf)
        pltpu.sync_copy(zbuf, sa_out.at[osl, :])
        pltpu.sync_copy(accew.at[ssl, :], zbuf)
        pltpu.sync_copy(zbuf, sew_out.at[osl, :])


def _tc_scalars_body(deg_ref, cnt_ref, dinv_ref, idg_ref, has_ref):
    dg = jnp.clip(deg_ref[...], 1e-9, None)
    dinv_ref[...] = lax.rsqrt(dg)
    idg_ref[...] = 1.0 / dg
    has_ref[...] = (cnt_ref[...] > 0).astype(jnp.float32)


def _tc_mix_body(wts_ref, eps_ref, x_ref, sa_ref, sew_ref, s1_ref, m_ref,
                 dinv_ref, idg_ref, has_ref,
                 wgcn, bgcn, wss, wsn, bsage, wms, wmn, bsmax,
                 wg1, bg1, wg2, bg2, wlin, blin, out_ref):
    w0 = wts_ref[0]
    w1 = wts_ref[1]
    w2 = wts_ref[2]
    w3 = wts_ref[3]
    w4 = wts_ref[4]
    ep = eps_ref[0]
    xv = x_ref[...]
    f32 = jnp.float32
    gcn = jnp.dot(dinv_ref[...] * sa_ref[...], wgcn[...],
                  preferred_element_type=f32) + bgcn[...]
    sage = (jnp.dot(xv, wss[...], preferred_element_type=f32)
            + jnp.dot(idg_ref[...] * sew_ref[...], wsn[...],
                      preferred_element_type=f32)
            + bsage[...])
    mxv = jnp.where(has_ref[...] > 0, m_ref[...], 0.0)
    smax = (jnp.dot(xv, wms[...], preferred_element_type=f32)
            + jnp.dot(mxv, wmn[...], preferred_element_type=f32) + bsmax[...])
    hg = (1.0 + ep) * xv + s1_ref[...]
    h1 = jax.nn.relu(jnp.dot(hg, wg1[...], preferred_element_type=f32)
                     + bg1[...])
    gin = jnp.dot(h1, wg2[...], preferred_element_type=f32) + bg2[...]
    lin = jnp.dot(xv, wlin[...], preferred_element_type=f32) + blin[...]
    out_ref[...] = w0 * gcn + w1 * sage + w2 * smax + w3 * gin + w4 * lin


@functools.cache
def _build_sc_kernels():
    mesh = plsc.VectorSubcoreMesh(core_axis_name="c", subcore_axis_name="s",
                                  num_cores=2, num_subcores=16)
    cparams = pltpu.CompilerParams(needs_layout_passes=False)

    sc_filter = pl.kernel(
        _sc_filter_body,
        out_type=[
            jax.ShapeDtypeStruct((NT * CAP,), jnp.int32),
            jax.ShapeDtypeStruct((NT * CAP,), jnp.float32),
            jax.ShapeDtypeStruct((NT * CAP,), jnp.int32),
            jax.ShapeDtypeStruct((NT * 16,), jnp.int32),
            jax.ShapeDtypeStruct((NPAD,), jnp.float32),
            jax.ShapeDtypeStruct((NPAD,), jnp.float32),
        ],
        mesh=mesh,
        compiler_params=cparams,
        scratch_types=[
            pltpu.VMEM((CAPB,), jnp.int32),
            pltpu.VMEM((CAPB,), jnp.float32),
            pltpu.VMEM((CAPB,), jnp.int32),
            pltpu.VMEM((NPT,), jnp.float32),
            pltpu.VMEM((NPT,), jnp.float32),
            pltpu.VMEM((2 * CH,), jnp.int32),
            pltpu.VMEM((2 * CH,), jnp.int32),
            pltpu.VMEM((2 * CH,), jnp.float32),
            pltpu.VMEM((16,), jnp.int32),
            pltpu.SemaphoreType.DMA,
            pltpu.SemaphoreType.DMA,
        ],
    )

    sc_max_s1 = pl.kernel(
        _sc_max_s1_body,
        out_type=[
            jax.ShapeDtypeStruct((NPAD * D,), jnp.float32),
            jax.ShapeDtypeStruct((NPAD, D), jnp.float32),
        ],
        mesh=mesh,
        compiler_params=cparams,
        scratch_types=[
            pltpu.VMEM((2, GB2 + 16), jnp.int32),
            pltpu.VMEM((2, GB2 + 16), jnp.int32),
            pltpu.VMEM((NPT * D + D,), jnp.float32),
            pltpu.VMEM((2, GB2, D), jnp.float32),
            pltpu.VMEM((2, GB2), jnp.int32),
            pltpu.VMEM((16,), jnp.int32),
            pltpu.VMEM((40, D), jnp.float32),
            pltpu.VMEM_SHARED((ACC_ROWS, D), jnp.float32),
            pltpu.SemaphoreType.DMA,
            pltpu.SemaphoreType.DMA,
            pltpu.SemaphoreType.DMA,
            pltpu.SemaphoreType.DMA,
        ],
    )

    sc_sums = pl.kernel(
        _sc_sums_body,
        out_type=[
            jax.ShapeDtypeStruct((NPAD, D), jnp.float32),
            jax.ShapeDtypeStruct((NPAD, D), jnp.float32),
        ],
        mesh=mesh,
        compiler_params=cparams,
        scratch_types=[
            pltpu.VMEM((2, GBB + 16), jnp.int32),
            pltpu.VMEM((2, GBB + 16), jnp.float32),
            pltpu.VMEM((2, GBB + 16), jnp.int32),
            pltpu.VMEM((2, GBB + 16), jnp.float32),
            pltpu.VMEM((2, GBB, D), jnp.float32),
            pltpu.VMEM((GBB, D), jnp.float32),
            pltpu.VMEM((GBB, D), jnp.float32),
            pltpu.VMEM((2, GBB), jnp.int32),
            pltpu.VMEM((2, GBB + 16), jnp.float32),
            pltpu.VMEM((16,), jnp.int32),
            pltpu.VMEM((40, D), jnp.float32),
            pltpu.VMEM_SHARED((ACC_ROWS, D), jnp.float32),
            pltpu.VMEM_SHARED((ACC_ROWS, D), jnp.float32),
            pltpu.SemaphoreType.DMA,
            pltpu.SemaphoreType.DMA,
            pltpu.SemaphoreType.DMA,
        ],
    )
    return sc_filter, sc_max_s1, sc_sums


def kernel(x, weights, edge_index, edge_weights, edge_attr, W_gcn, b_gcn,
           W_sage_self, W_sage_neigh, b_sage, W_smax_self, W_smax_neigh,
           b_smax, W_gin1, b_gin1, W_gin2, b_gin2, eps, W_lin, b_lin,
           with_linear):
    n, d = x.shape
    xp = jnp.pad(x, ((0, NPAD - n), (0, 0)))
    src = edge_index[0]
    dst = edge_index[1]

    sc_filter, sc_max_s1, sc_sums = _build_sc_kernels()
    srcl, ewl, dstll, ecnt, deg, cntv = sc_filter(src, dst, edge_weights)
    mxf, s1 = sc_max_s1(srcl, dstll, ecnt, xp)

    dinv2, idg2, has2 = pl.pallas_call(
        _tc_scalars_body,
        grid=(1,),
        in_specs=[pl.BlockSpec((NPAD // D, D), lambda i: (0, 0)),
                  pl.BlockSpec((NPAD // D, D), lambda i: (0, 0))],
        out_specs=[pl.BlockSpec((NPAD // D, D), lambda i: (0, 0))] * 3,
        out_shape=[jax.ShapeDtypeStruct((NPAD // D, D), jnp.float32)] * 3,
    )(deg.reshape(NPAD // D, D), cntv.reshape(NPAD // D, D))

    sa, sew = sc_sums(srcl, ewl, dstll, ecnt, dinv2.reshape(NPAD), xp)

    nblk = NPAD // D
    row_spec = pl.BlockSpec((D, D), lambda i: (i, 0))
    col_spec = pl.BlockSpec((D, 1), lambda i: (i, 0))
    full_spec = pl.BlockSpec((D, D), lambda i: (0, 0))
    bias_spec = pl.BlockSpec((1, D), lambda i: (0, 0))
    smem_spec = pl.BlockSpec(memory_space=pltpu.SMEM)
    out = pl.pallas_call(
        _tc_mix_body,
        grid=(nblk,),
        in_specs=[smem_spec, smem_spec,
                  row_spec, row_spec, row_spec, row_spec, row_spec,
                  col_spec, col_spec, col_spec,
                  full_spec, bias_spec, full_spec, full_spec, bias_spec,
                  full_spec, full_spec, bias_spec,
                  full_spec, bias_spec, full_spec, bias_spec,
                  full_spec, bias_spec],
        out_specs=pl.BlockSpec((D, D), lambda i: (i, 0)),
        out_shape=jax.ShapeDtypeStruct((NPAD, D), jnp.float32),
    )(weights, eps.reshape(1), xp, sa, sew, s1, mxf.reshape(NPAD, D),
      dinv2.reshape(NPAD, 1), idg2.reshape(NPAD, 1), has2.reshape(NPAD, 1),
      W_gcn, b_gcn.reshape(1, D), W_sage_self, W_sage_neigh,
      b_sage.reshape(1, D), W_smax_self, W_smax_neigh, b_smax.reshape(1, D),
      W_gin1, b_gin1.reshape(1, D), W_gin2, b_gin2.reshape(1, D),
      W_lin, b_lin.reshape(1, D))
    return out[:n]
